# baseline (speedup 1.0000x reference)
import functools

import jax
import jax.numpy as jnp
from jax import lax
from jax.experimental import pallas as pl
from jax.experimental.pallas import tpu as pltpu
from jax.experimental.pallas import tpu_sc as plsc

NUM_CORES = 2
NUM_SUBCORES = 16
NUM_WORKERS = NUM_CORES * NUM_SUBCORES
LANES = 16

ROWS_PER_GROUP = 4
CHUNK = 4096


def _build_sc_call(batch, input_size, total_syn):
    assert batch % (NUM_WORKERS * ROWS_PER_GROUP) == 0
    assert total_syn % CHUNK == 0 and CHUNK % LANES == 0
    rows_per_worker = batch // NUM_WORKERS
    groups = rows_per_worker // ROWS_PER_GROUP
    chunks = total_syn // CHUNK

    mesh = plsc.VectorSubcoreMesh(
        core_axis_name="c", subcore_axis_name="s", num_cores=NUM_CORES
    )

    @functools.partial(
        pl.kernel,
        out_type=jax.ShapeDtypeStruct((batch, total_syn), jnp.float32),
        mesh=mesh,
        compiler_params=pltpu.CompilerParams(needs_layout_passes=False),
        scratch_types=[
            *[pltpu.VMEM((input_size,), jnp.float32) for _ in range(ROWS_PER_GROUP)],
            pltpu.VMEM((CHUNK,), jnp.int32),
            pltpu.VMEM((CHUNK,), jnp.int32),
            *[pltpu.VMEM((CHUNK,), jnp.float32) for _ in range(ROWS_PER_GROUP)],
        ],
    )
    def sc_call(x_hbm, ip_hbm, iq_hbm, out_hbm, r0, r1, r2, r3, ipv, iqv,
                o0, o1, o2, o3):
        rows = [r0, r1, r2, r3]
        outs = [o0, o1, o2, o3]
        wid = lax.axis_index("s") * NUM_CORES + lax.axis_index("c")
        base = wid * rows_per_worker

        def group_body(g, carry):
            row0 = base + g * ROWS_PER_GROUP
            for r in range(ROWS_PER_GROUP):
                pltpu.sync_copy(x_hbm.at[row0 + r], rows[r])

            def chunk_body(c, carry):
                off = c * CHUNK
                pltpu.sync_copy(ip_hbm.at[pl.ds(off, CHUNK)], ipv)
                pltpu.sync_copy(iq_hbm.at[pl.ds(off, CHUNK)], iqv)

                def vec_body(i, carry):
                    ip = ipv[pl.ds(i * LANES, LANES)]
                    iq = iqv[pl.ds(i * LANES, LANES)]
                    for r in range(ROWS_PER_GROUP):
                        u = plsc.load_gather(rows[r], [ip])
                        v = plsc.load_gather(rows[r], [iq])
                        outs[r][pl.ds(i * LANES, LANES)] = jnp.where(
                            u > v, jnp.float32(1.0), jnp.float32(0.0)
                        )
                    return carry

                lax.fori_loop(0, CHUNK // LANES, vec_body, 0, unroll=False)
                for r in range(ROWS_PER_GROUP):
                    pltpu.sync_copy(
                        outs[r], out_hbm.at[row0 + r, pl.ds(off, CHUNK)]
                    )
                return carry

            return lax.fori_loop(0, chunks, chunk_body, carry, unroll=False)

        lax.fori_loop(0, groups, group_body, 0, unroll=False)

    return sc_call


def kernel(x, idx_p, idx_q):
    batch, input_size = x.shape
    num_neurons, num_synapses = idx_p.shape
    total_syn = num_neurons * num_synapses
    ip = idx_p.reshape(total_syn).astype(jnp.int32)
    iq = idx_q.reshape(total_syn).astype(jnp.int32)
    sc_call = _build_sc_call(batch, input_size, total_syn)
    out = sc_call(x, ip, iq)
    return out.reshape(batch, num_neurons, num_synapses)
